# Initial kernel scaffold; baseline (speedup 1.0000x reference)
import jax
import jax.numpy as jnp
from jax.experimental import pallas as pl


def kernel(inputs, roi_label, lut, cq, cqb):
    raise NotImplementedError("write your pallas kernel here")


# speedup vs baseline: 1.6884x; 1.6884x over previous
import functools

import jax
import jax.numpy as jnp
from jax.experimental import pallas as pl
from jax.experimental.pallas import tpu as pltpu

_NF = 128
_NP = 100000
_NCQ = 5000
_NBG = 5000
_B = 1024
_SCALAR = 30.0
_AD = 0.25
_AR = 0.25
_BLK = 5000
_KLUT = _NP // _BLK
_T = _KLUT + 1
_PREC = jax.lax.Precision.HIGHEST


def _mm(w, xt):
    return jax.lax.dot_general(
        w, xt, (((1,), (0,)), ((), ())),
        preferred_element_type=jnp.float32, precision=_PREC)


def _body(xt_ref, cqb_ref, lut_ref, cq_ref, lab_ref, roi_ref,
          cls_ref, det_ref, oim_ref, zb_ref, zn_ref, sl_ref):
    g = pl.program_id(0)

    @pl.when(g == 0)
    def _init():
        sb = _mm(cqb_ref[...], xt_ref[...]) * _SCALAR
        zb_ref[...] = jnp.sum(jnp.exp(sb - _SCALAR), axis=0, keepdims=True)
        sc = _mm(cq_ref[...], xt_ref[...]) * _SCALAR
        zn_ref[...] = jnp.sum(jnp.exp(sc - _SCALAR), axis=0, keepdims=True)
        sl_ref[...] = jnp.zeros((1, _B), jnp.float32)

    @pl.when(g > 0)
    def _lut_step():
        s = _mm(lut_ref[...], xt_ref[...]) * _SCALAR
        zn_ref[...] += jnp.sum(jnp.exp(s - _SCALAR), axis=0, keepdims=True)
        base = (g - 1) * _BLK
        rows = base + jax.lax.broadcasted_iota(jnp.int32, (_BLK, _B), 0)
        hit = rows == lab_ref[...]
        sl_ref[...] += jnp.sum(jnp.where(hit, s, 0.0), axis=0, keepdims=True)

    @pl.when(g == _T - 1)
    def _finalize():
        zb = zb_ref[...]
        zn = zn_ref[...]
        tot = zb + zn
        c0 = zb / tot
        c1 = zn / tot
        roi = roi_ref[...]
        c_det = jnp.where(roi == -2, c0, c1)
        f_det = _AD * (1.0 - c_det) ** 2.0 * jnp.log(c_det)
        det_ref[...] = -jnp.sum(f_det, axis=1, keepdims=True) / float(_B)
        p = jnp.exp(sl_ref[...] - _SCALAR) / zn
        per = -_AR * (1.0 - p) ** 2.0 * jnp.log(p)
        validf = (roi >= 0).astype(jnp.float32)
        maskf = (roi >= -1).astype(jnp.float32)
        n_valid = jnp.maximum(jnp.sum(maskf, axis=1, keepdims=True), 1.0)
        oim_vec = per * validf * c1 * c1
        oim_ref[...] = jnp.sum(oim_vec, axis=1, keepdims=True) / n_valid
        cls_ref[...] = jnp.concatenate([c0, c1], axis=0)


@functools.partial(jax.jit, static_argnames=())
def _run(xt, cqb, lut, cq, lab, roi):
    return pl.pallas_call(
        _body,
        grid=(_T,),
        in_specs=[
            pl.BlockSpec((_NF, _B), lambda g: (0, 0)),
            pl.BlockSpec((_NBG, _NF), lambda g: (0, 0)),
            pl.BlockSpec((_BLK, _NF), lambda g: (jnp.clip(g - 1, 0, _KLUT - 1), 0)),
            pl.BlockSpec((_NCQ, _NF), lambda g: (0, 0)),
            pl.BlockSpec((1, _B), lambda g: (0, 0)),
            pl.BlockSpec((1, _B), lambda g: (0, 0)),
        ],
        out_specs=[
            pl.BlockSpec((2, _B), lambda g: (0, 0)),
            pl.BlockSpec((1, 1), lambda g: (0, 0)),
            pl.BlockSpec((1, 1), lambda g: (0, 0)),
        ],
        out_shape=[
            jax.ShapeDtypeStruct((2, _B), jnp.float32),
            jax.ShapeDtypeStruct((1, 1), jnp.float32),
            jax.ShapeDtypeStruct((1, 1), jnp.float32),
        ],
        scratch_shapes=[
            pltpu.VMEM((1, _B), jnp.float32),
            pltpu.VMEM((1, _B), jnp.float32),
            pltpu.VMEM((1, _B), jnp.float32),
        ],
    )(xt, cqb, lut, cq, lab, roi)


def kernel(inputs, roi_label, lut, cq, cqb):
    xt = inputs.T
    roi = roi_label.astype(jnp.int32).reshape(1, _B)
    lab = jnp.clip(roi, 0, _NP - 1)
    cls_t, det, oim = _run(xt, cqb, lut, cq, lab, roi)
    return cls_t.T, det.reshape(()), oim.reshape(())


# speedup vs baseline: 13.2621x; 7.8547x over previous
import functools

import jax
import jax.numpy as jnp
from jax.experimental import pallas as pl
from jax.experimental.pallas import tpu as pltpu

_NF = 128
_NP = 100000
_NCQ = 5000
_NBG = 5000
_B = 1024
_SCALAR = 30.0
_AD = 0.25
_AR = 0.25
_BLK = 5000
_KLUT = _NP // _BLK
_T = _KLUT + 1
_PREC = jax.lax.Precision.DEFAULT


def _mm(w, xt):
    return jax.lax.dot_general(
        w.astype(jnp.bfloat16), xt.astype(jnp.bfloat16), (((1,), (0,)), ((), ())),
        preferred_element_type=jnp.float32, precision=_PREC)


def _body(xt_ref, cqb_ref, lut_ref, cq_ref, lab_ref, roi_ref,
          cls_ref, det_ref, oim_ref, zb_ref, zn_ref, sl_ref):
    g = pl.program_id(0)

    @pl.when(g == 0)
    def _init():
        sb = _mm(cqb_ref[...], xt_ref[...]) * _SCALAR
        zb_ref[...] = jnp.sum(jnp.exp(sb - _SCALAR), axis=0, keepdims=True)
        sc = _mm(cq_ref[...], xt_ref[...]) * _SCALAR
        zn_ref[...] = jnp.sum(jnp.exp(sc - _SCALAR), axis=0, keepdims=True)
        sl_ref[...] = jnp.zeros((1, _B), jnp.float32)

    @pl.when(g > 0)
    def _lut_step():
        s = _mm(lut_ref[...], xt_ref[...]) * _SCALAR
        zn_ref[...] += jnp.sum(jnp.exp(s - _SCALAR), axis=0, keepdims=True)
        base = (g - 1) * _BLK
        rows = base + jax.lax.broadcasted_iota(jnp.int32, (_BLK, _B), 0)
        hit = rows == lab_ref[...]
        sl_ref[...] += jnp.sum(jnp.where(hit, s, 0.0), axis=0, keepdims=True)

    @pl.when(g == _T - 1)
    def _finalize():
        zb = zb_ref[...]
        zn = zn_ref[...]
        tot = zb + zn
        c0 = zb / tot
        c1 = zn / tot
        roi = roi_ref[...]
        c_det = jnp.where(roi == -2, c0, c1)
        f_det = _AD * (1.0 - c_det) ** 2.0 * jnp.log(c_det)
        det_ref[...] = -jnp.sum(f_det, axis=1, keepdims=True) / float(_B)
        p = jnp.exp(sl_ref[...] - _SCALAR) / zn
        per = -_AR * (1.0 - p) ** 2.0 * jnp.log(p)
        validf = (roi >= 0).astype(jnp.float32)
        maskf = (roi >= -1).astype(jnp.float32)
        n_valid = jnp.maximum(jnp.sum(maskf, axis=1, keepdims=True), 1.0)
        oim_vec = per * validf * c1 * c1
        oim_ref[...] = jnp.sum(oim_vec, axis=1, keepdims=True) / n_valid
        cls_ref[...] = jnp.concatenate([c0, c1], axis=0)


@functools.partial(jax.jit, static_argnames=())
def _run(xt, cqb, lut, cq, lab, roi):
    return pl.pallas_call(
        _body,
        grid=(_T,),
        in_specs=[
            pl.BlockSpec((_NF, _B), lambda g: (0, 0)),
            pl.BlockSpec((_NBG, _NF), lambda g: (0, 0)),
            pl.BlockSpec((_BLK, _NF), lambda g: (jnp.clip(g - 1, 0, _KLUT - 1), 0)),
            pl.BlockSpec((_NCQ, _NF), lambda g: (0, 0)),
            pl.BlockSpec((1, _B), lambda g: (0, 0)),
            pl.BlockSpec((1, _B), lambda g: (0, 0)),
        ],
        out_specs=[
            pl.BlockSpec((2, _B), lambda g: (0, 0)),
            pl.BlockSpec((1, 1), lambda g: (0, 0)),
            pl.BlockSpec((1, 1), lambda g: (0, 0)),
        ],
        out_shape=[
            jax.ShapeDtypeStruct((2, _B), jnp.float32),
            jax.ShapeDtypeStruct((1, 1), jnp.float32),
            jax.ShapeDtypeStruct((1, 1), jnp.float32),
        ],
        scratch_shapes=[
            pltpu.VMEM((1, _B), jnp.float32),
            pltpu.VMEM((1, _B), jnp.float32),
            pltpu.VMEM((1, _B), jnp.float32),
        ],
    )(xt, cqb, lut, cq, lab, roi)


def kernel(inputs, roi_label, lut, cq, cqb):
    xt = inputs.T
    roi = roi_label.astype(jnp.int32).reshape(1, _B)
    lab = jnp.clip(roi, 0, _NP - 1)
    cls_t, det, oim = _run(xt, cqb, lut, cq, lab, roi)
    return cls_t.T, det.reshape(()), oim.reshape(())


# speedup vs baseline: 16.6725x; 1.2572x over previous
import functools
import math

import jax
import jax.numpy as jnp
from jax.experimental import pallas as pl
from jax.experimental.pallas import tpu as pltpu

_NF = 128
_NP = 100000
_NCQ = 5000
_NBG = 5000
_B = 1024
_SCALAR = 30.0
_AD = 0.25
_AR = 0.25
_BLK = 5000
_KLUT = _NP // _BLK
_T = _KLUT + 1


_C = _SCALAR / math.log(2.0)


def _expmm(w, xt):
    d = jax.lax.dot_general(
        w.astype(jnp.bfloat16), xt, (((1,), (0,)), ((), ())),
        preferred_element_type=jnp.float32)
    return jnp.exp2(d * _C)


def _body(xt_ref, cqb_ref, lut_ref, cq_ref, lab_ref, roi_ref,
          cls_ref, det_ref, oim_ref, zb_ref, zn_ref, el_ref):
    g = pl.program_id(0)

    @pl.when(g == 0)
    def _init():
        eb = _expmm(cqb_ref[...], xt_ref[...])
        zb_ref[...] = jnp.sum(eb, axis=0, keepdims=True)
        ec = _expmm(cq_ref[...], xt_ref[...])
        zn_ref[...] = jnp.sum(ec, axis=0, keepdims=True)
        el_ref[...] = jnp.zeros((1, _B), jnp.float32)

    @pl.when(g > 0)
    def _lut_step():
        e = _expmm(lut_ref[...], xt_ref[...])
        zn_ref[...] += jnp.sum(e, axis=0, keepdims=True)
        base = (g - 1) * _BLK
        rows = base + jax.lax.broadcasted_iota(jnp.int32, (_BLK, _B), 0)
        hit = rows == lab_ref[...]
        el_ref[...] += jnp.sum(jnp.where(hit, e, 0.0), axis=0, keepdims=True)

    @pl.when(g == _T - 1)
    def _finalize():
        zb = zb_ref[...]
        zn = zn_ref[...]
        tot = zb + zn
        c0 = zb / tot
        c1 = zn / tot
        roi = roi_ref[...]
        c_det = jnp.where(roi == -2, c0, c1)
        f_det = _AD * (1.0 - c_det) ** 2.0 * jnp.log(c_det)
        det_ref[...] = -jnp.sum(f_det, axis=1, keepdims=True) / float(_B)
        p = el_ref[...] / zn
        per = -_AR * (1.0 - p) ** 2.0 * jnp.log(p)
        validf = (roi >= 0).astype(jnp.float32)
        maskf = (roi >= -1).astype(jnp.float32)
        n_valid = jnp.maximum(jnp.sum(maskf, axis=1, keepdims=True), 1.0)
        oim_vec = per * validf * c1 * c1
        oim_ref[...] = jnp.sum(oim_vec, axis=1, keepdims=True) / n_valid
        cls_ref[...] = jnp.concatenate([c0, c1], axis=0)


@functools.partial(jax.jit, static_argnames=())
def _run(xt, cqb, lut, cq, lab, roi):
    return pl.pallas_call(
        _body,
        grid=(_T,),
        in_specs=[
            pl.BlockSpec((_NF, _B), lambda g: (0, 0)),
            pl.BlockSpec((_NBG, _NF), lambda g: (0, 0)),
            pl.BlockSpec((_BLK, _NF), lambda g: (jnp.clip(g - 1, 0, _KLUT - 1), 0)),
            pl.BlockSpec((_NCQ, _NF), lambda g: (0, 0)),
            pl.BlockSpec((1, _B), lambda g: (0, 0)),
            pl.BlockSpec((1, _B), lambda g: (0, 0)),
        ],
        out_specs=[
            pl.BlockSpec((2, _B), lambda g: (0, 0)),
            pl.BlockSpec((1, 1), lambda g: (0, 0)),
            pl.BlockSpec((1, 1), lambda g: (0, 0)),
        ],
        out_shape=[
            jax.ShapeDtypeStruct((2, _B), jnp.float32),
            jax.ShapeDtypeStruct((1, 1), jnp.float32),
            jax.ShapeDtypeStruct((1, 1), jnp.float32),
        ],
        scratch_shapes=[
            pltpu.VMEM((1, _B), jnp.float32),
            pltpu.VMEM((1, _B), jnp.float32),
            pltpu.VMEM((1, _B), jnp.float32),
        ],
    )(xt, cqb, lut, cq, lab, roi)


def kernel(inputs, roi_label, lut, cq, cqb):
    xt = inputs.T.astype(jnp.bfloat16)
    roi = roi_label.astype(jnp.int32).reshape(1, _B)
    lab = jnp.clip(roi, 0, _NP - 1)
    cls_t, det, oim = _run(xt, cqb, lut, cq, lab, roi)
    return cls_t.T, det.reshape(()), oim.reshape(())
